# Initial kernel scaffold; baseline (speedup 1.0000x reference)
import jax
import jax.numpy as jnp
from jax.experimental import pallas as pl


def kernel(x, emb, W1, b1, W2, b2, Wout, bout):
    raise NotImplementedError("write your pallas kernel here")


# speedup vs baseline: 48.7638x; 48.7638x over previous
import functools

import jax
import jax.numpy as jnp
from jax import lax
from jax.experimental import pallas as pl
from jax.experimental.pallas import tpu as pltpu
from jax.experimental.pallas import tpu_sc as plsc

_VOCAB = 1000
_VP = 1024
_EMB = 64
_B = 16384
_L = 200
_LP = 208
_H1 = 256
_H2 = 256
_OUT = 128

_NW = 32
_ROWS_PER_W = _B // _NW
_CHUNK = 64
_NCHUNK = _ROWS_PER_W // _CHUNK
_VREGS_PER_ROW = _LP // 16


def _counts_body(x_hbm, cnt_hbm, x_v, cnt_v):
    wid = lax.axis_index("s") * 2 + lax.axis_index("c")

    def chunk_body(c, carry):
        rowbase = wid * _ROWS_PER_W + c * _CHUNK
        pltpu.sync_copy(x_hbm.at[pl.ds(rowbase * _LP, _CHUNK * _LP)], x_v)

        zero16 = jnp.zeros((16,), jnp.float32)

        def zbody(i, carry):
            base = i * 64
            cnt_v[pl.ds(base, 16)] = zero16
            cnt_v[pl.ds(base + 16, 16)] = zero16
            cnt_v[pl.ds(base + 32, 16)] = zero16
            cnt_v[pl.ds(base + 48, 16)] = zero16
            return carry

        lax.fori_loop(0, _CHUNK * _VP // 64, zbody, 0)

        ones16 = jnp.ones((16,), jnp.float32)

        def rbody(r, carry):
            xbase = r * _LP
            cbase = r * _VP

            def jbody(j, carry):
                xv = x_v[pl.ds(xbase + j * 16, 16)]
                plsc.addupdate_scatter(cnt_v, [xv + cbase], ones16)
                return carry

            return lax.fori_loop(0, _VREGS_PER_ROW, jbody, carry)

        lax.fori_loop(0, _CHUNK, rbody, 0)

        pltpu.sync_copy(cnt_v, cnt_hbm.at[pl.ds(rowbase * _VP, _CHUNK * _VP)])
        return carry

    lax.fori_loop(0, _NCHUNK, chunk_body, 0)


_counts_call = pl.kernel(
    _counts_body,
    out_type=jax.ShapeDtypeStruct((_B * _VP,), jnp.float32),
    mesh=plsc.VectorSubcoreMesh(core_axis_name="c", subcore_axis_name="s"),
    scratch_types=[
        pltpu.VMEM((_CHUNK * _LP,), jnp.int32),
        pltpu.VMEM((_CHUNK * _VP,), jnp.float32),
    ],
    compiler_params=pltpu.CompilerParams(needs_layout_passes=False),
)

_BB = 512


def _mlp_body(cnt, embp, w1, b1, w2, b2, wout, bout, out, m1):
    @pl.when(pl.program_id(0) == 0)
    def _():
        m1[...] = jnp.dot(embp[...], w1[...],
                          preferred_element_type=jnp.float32) * (1.0 / _L)

    h = jnp.dot(cnt[...], m1[...], preferred_element_type=jnp.float32)
    h = jnp.maximum(h + b1[...], 0.0)
    h = jnp.maximum(
        jnp.dot(h, w2[...], preferred_element_type=jnp.float32) + b2[...], 0.0)
    out[...] = jnp.dot(h, wout[...],
                       preferred_element_type=jnp.float32) + bout[...]


_mlp_call = pl.pallas_call(
    _mlp_body,
    grid=(_B // _BB,),
    in_specs=[
        pl.BlockSpec((_BB, _VP), lambda i: (i, 0)),
        pl.BlockSpec((_VP, _EMB), lambda i: (0, 0)),
        pl.BlockSpec((_EMB, _H1), lambda i: (0, 0)),
        pl.BlockSpec((1, _H1), lambda i: (0, 0)),
        pl.BlockSpec((_H1, _H2), lambda i: (0, 0)),
        pl.BlockSpec((1, _H2), lambda i: (0, 0)),
        pl.BlockSpec((_H2, _OUT), lambda i: (0, 0)),
        pl.BlockSpec((1, _OUT), lambda i: (0, 0)),
    ],
    out_specs=pl.BlockSpec((_BB, _OUT), lambda i: (i, 0)),
    out_shape=jax.ShapeDtypeStruct((_B, _OUT), jnp.float32),
    scratch_shapes=[pltpu.VMEM((_VP, _H1), jnp.float32)],
)


def kernel(x, emb, W1, b1, W2, b2, Wout, bout):
    xp = jnp.pad(x, ((0, 0), (0, _LP - _L)),
                 constant_values=_VOCAB).reshape(-1)
    embp = jnp.pad(emb, ((0, _VP - _VOCAB), (0, 0)))
    cnt = _counts_call(xp).reshape(_B, _VP)
    return _mlp_call(cnt, embp, W1, b1.reshape(1, _H1), W2,
                     b2.reshape(1, _H2), Wout, bout.reshape(1, _OUT))


# speedup vs baseline: 67.5460x; 1.3852x over previous
import jax
import jax.numpy as jnp
from jax import lax
from jax.experimental import pallas as pl
from jax.experimental.pallas import tpu as pltpu
from jax.experimental.pallas import tpu_sc as plsc

_VOCAB = 1000
_VP = 1024
_EMB = 64
_B = 16384
_L = 200
_H1 = 256
_H2 = 256
_OUT = 128

_NW = 32
_ROWS_PER_W = _B // _NW
_CHUNK = 64
_NCHUNK = _ROWS_PER_W // _CHUNK
_JFULL = _L // 16


def _counts_body(x_hbm, cnt_hbm, x_v, cnt_v):
    wid = lax.axis_index("s") * 2 + lax.axis_index("c")
    lanes = lax.iota(jnp.int32, 16)
    tail_mask = lanes >= 8
    ones16 = jnp.ones((16,), jnp.float32)
    zero16 = jnp.zeros((16,), jnp.float32)

    def chunk_body(c, carry):
        rowbase = wid * _ROWS_PER_W + c * _CHUNK
        pltpu.sync_copy(x_hbm.at[pl.ds(rowbase, _CHUNK)], x_v)

        def zbody(r, carry):
            def zcol(j, carry):
                base = j * 64
                cnt_v[r, pl.ds(base, 16)] = zero16
                cnt_v[r, pl.ds(base + 16, 16)] = zero16
                cnt_v[r, pl.ds(base + 32, 16)] = zero16
                cnt_v[r, pl.ds(base + 48, 16)] = zero16
                return carry
            return lax.fori_loop(0, _VP // 64, zcol, carry)

        lax.fori_loop(0, _CHUNK, zbody, 0)

        def rbody(r, carry):
            rvec = jnp.full((16,), r, jnp.int32)

            def jbody(j, carry):
                xv = x_v[r, pl.ds(j * 16, 16)]
                plsc.addupdate_scatter(cnt_v, [rvec, xv], ones16)
                return carry

            lax.fori_loop(0, _JFULL, jbody, carry)
            xt = x_v[r, pl.ds(_L - 16, 16)]
            plsc.addupdate_scatter(cnt_v, [rvec, xt], ones16, mask=tail_mask)
            return carry

        lax.fori_loop(0, _CHUNK, rbody, 0)

        pltpu.sync_copy(cnt_v, cnt_hbm.at[pl.ds(rowbase, _CHUNK)])
        return carry

    lax.fori_loop(0, _NCHUNK, chunk_body, 0)


_counts_call = pl.kernel(
    _counts_body,
    out_type=jax.ShapeDtypeStruct((_B, _VP), jnp.float32),
    mesh=plsc.VectorSubcoreMesh(core_axis_name="c", subcore_axis_name="s"),
    scratch_types=[
        pltpu.VMEM((_CHUNK, _L), jnp.int32),
        pltpu.VMEM((_CHUNK, _VP), jnp.float32),
    ],
    compiler_params=pltpu.CompilerParams(needs_layout_passes=False),
)

_BB = 512


def _mlp_body(cnt, embp, w1, b1, w2, b2, wout, bout, out, m1):
    @pl.when(pl.program_id(0) == 0)
    def _():
        m1[...] = jnp.dot(embp[...], w1[...],
                          preferred_element_type=jnp.float32) * (1.0 / _L)

    h = jnp.dot(cnt[...], m1[...], preferred_element_type=jnp.float32)
    h = jnp.maximum(h + b1[...], 0.0)
    h = jnp.maximum(
        jnp.dot(h, w2[...], preferred_element_type=jnp.float32) + b2[...], 0.0)
    out[...] = jnp.dot(h, wout[...],
                       preferred_element_type=jnp.float32) + bout[...]


_mlp_call = pl.pallas_call(
    _mlp_body,
    grid=(_B // _BB,),
    in_specs=[
        pl.BlockSpec((_BB, _VP), lambda i: (i, 0)),
        pl.BlockSpec((_VP, _EMB), lambda i: (0, 0)),
        pl.BlockSpec((_EMB, _H1), lambda i: (0, 0)),
        pl.BlockSpec((1, _H1), lambda i: (0, 0)),
        pl.BlockSpec((_H1, _H2), lambda i: (0, 0)),
        pl.BlockSpec((1, _H2), lambda i: (0, 0)),
        pl.BlockSpec((_H2, _OUT), lambda i: (0, 0)),
        pl.BlockSpec((1, _OUT), lambda i: (0, 0)),
    ],
    out_specs=pl.BlockSpec((_BB, _OUT), lambda i: (i, 0)),
    out_shape=jax.ShapeDtypeStruct((_B, _OUT), jnp.float32),
    scratch_shapes=[pltpu.VMEM((_VP, _H1), jnp.float32)],
)


def kernel(x, emb, W1, b1, W2, b2, Wout, bout):
    embp = jnp.pad(emb, ((0, _VP - _VOCAB), (0, 0)))
    cnt = _counts_call(x)
    return _mlp_call(cnt, embp, W1, b1.reshape(1, _H1), W2,
                     b2.reshape(1, _H2), Wout, bout.reshape(1, _OUT))


# speedup vs baseline: 93.4203x; 1.3831x over previous
import jax
import jax.numpy as jnp
from jax import lax
from jax.experimental import pallas as pl
from jax.experimental.pallas import tpu as pltpu
from jax.experimental.pallas import tpu_sc as plsc

_VOCAB = 1000
_VP = 1024
_EMB = 64
_B = 16384
_L = 200
_H1 = 256
_H2 = 256
_OUT = 128

_NW = 32
_ROWS_PER_W = _B // _NW
_CHUNK = 64
_NCHUNK = _ROWS_PER_W // _CHUNK
_JFULL = _L // 16


_WP = _VP // 4


def _counts_body(x_hbm, cnt_hbm, x_v, cnt_v):
    wid = lax.axis_index("s") * 2 + lax.axis_index("c")
    lanes = lax.iota(jnp.int32, 16)
    tail_mask = lanes >= 8
    one16 = jnp.full((16,), 1, jnp.int32)
    zero16 = jnp.zeros((16,), jnp.int32)

    def scat(cv, xv, rvec, mask=None):
        col = jnp.bitwise_and(xv, 255)
        sh = jnp.right_shift(xv, 5) & 24
        val = jnp.left_shift(one16, sh)
        plsc.addupdate_scatter(cv, [rvec, col], val, mask=mask)

    def chunk_body(c, carry):
        rowbase = wid * _ROWS_PER_W + c * _CHUNK
        pltpu.sync_copy(x_hbm.at[pl.ds(rowbase, _CHUNK)], x_v)

        def zbody(r, carry):
            for j in range(_WP // 16):
                cnt_v[r, pl.ds(j * 16, 16)] = zero16
            return carry

        lax.fori_loop(0, _CHUNK, zbody, 0)

        def rbody(r, carry):
            rvec = jnp.full((16,), r, jnp.int32)
            for j in range(_JFULL):
                scat(cnt_v, x_v[r, pl.ds(j * 16, 16)], rvec)
            scat(cnt_v, x_v[r, pl.ds(_L - 16, 16)], rvec, mask=tail_mask)
            return carry

        lax.fori_loop(0, _CHUNK, rbody, 0)

        pltpu.sync_copy(cnt_v, cnt_hbm.at[pl.ds(rowbase, _CHUNK)])
        return carry

    lax.fori_loop(0, _NCHUNK, chunk_body, 0)


_counts_call = pl.kernel(
    _counts_body,
    out_type=jax.ShapeDtypeStruct((_B, _WP), jnp.int32),
    mesh=plsc.VectorSubcoreMesh(core_axis_name="c", subcore_axis_name="s"),
    scratch_types=[
        pltpu.VMEM((_CHUNK, _L), jnp.int32),
        pltpu.VMEM((_CHUNK, _WP), jnp.int32),
    ],
    compiler_params=pltpu.CompilerParams(needs_layout_passes=False),
)

_BB = 512


def _mlp_body(cnt, embp, w1, b1, w2, b2, wout, bout, out, m1):
    @pl.when(pl.program_id(0) == 0)
    def _():
        m1[...] = jnp.dot(embp[...], w1[...],
                          preferred_element_type=jnp.float32) * (1.0 / _L)

    w = cnt[...]
    h = jnp.zeros((_BB, _H1), jnp.float32)
    for k in range(4):
        part = ((w >> (8 * k)) & 0xFF).astype(jnp.float32)
        h = h + jnp.dot(part, m1[pl.ds(k * _WP, _WP), :],
                        preferred_element_type=jnp.float32)
    h = jnp.maximum(h + b1[...], 0.0)
    h = jnp.maximum(
        jnp.dot(h, w2[...], preferred_element_type=jnp.float32) + b2[...], 0.0)
    out[...] = jnp.dot(h, wout[...],
                       preferred_element_type=jnp.float32) + bout[...]


_mlp_call = pl.pallas_call(
    _mlp_body,
    grid=(_B // _BB,),
    in_specs=[
        pl.BlockSpec((_BB, _WP), lambda i: (i, 0)),
        pl.BlockSpec((_VP, _EMB), lambda i: (0, 0)),
        pl.BlockSpec((_EMB, _H1), lambda i: (0, 0)),
        pl.BlockSpec((1, _H1), lambda i: (0, 0)),
        pl.BlockSpec((_H1, _H2), lambda i: (0, 0)),
        pl.BlockSpec((1, _H2), lambda i: (0, 0)),
        pl.BlockSpec((_H2, _OUT), lambda i: (0, 0)),
        pl.BlockSpec((1, _OUT), lambda i: (0, 0)),
    ],
    out_specs=pl.BlockSpec((_BB, _OUT), lambda i: (i, 0)),
    out_shape=jax.ShapeDtypeStruct((_B, _OUT), jnp.float32),
    scratch_shapes=[pltpu.VMEM((_VP, _H1), jnp.float32)],
)


def kernel(x, emb, W1, b1, W2, b2, Wout, bout):
    embp = jnp.pad(emb, ((0, _VP - _VOCAB), (0, 0)))
    cnt = _counts_call(x)
    return _mlp_call(cnt, embp, W1, b1.reshape(1, _H1), W2,
                     b2.reshape(1, _H2), Wout, bout.reshape(1, _OUT))


# speedup vs baseline: 127.2561x; 1.3622x over previous
import jax
import jax.numpy as jnp
from jax import lax
from jax.experimental import pallas as pl
from jax.experimental.pallas import tpu as pltpu
from jax.experimental.pallas import tpu_sc as plsc

_VOCAB = 1000
_VP = 1024
_EMB = 64
_B = 16384
_L = 200
_H1 = 256
_H2 = 256
_OUT = 128

_NW = 32
_ROWS_PER_W = _B // _NW
_CHUNK = 64
_NCHUNK = _ROWS_PER_W // _CHUNK
_JFULL = _L // 16


_WP = _VP // 4


def _counts_body(x_hbm, cnt_hbm, x_v, cnt_v):
    wid = lax.axis_index("s") * 2 + lax.axis_index("c")
    lanes = lax.iota(jnp.int32, 16)
    tail_mask = lanes >= 8
    one16 = jnp.full((16,), 1, jnp.int32)
    zero16 = jnp.zeros((16,), jnp.int32)

    def scat(cv, xv, rvec, mask=None):
        col = jnp.bitwise_and(xv, 255)
        sh = jnp.right_shift(xv, 5) & 24
        val = jnp.left_shift(one16, sh)
        plsc.addupdate_scatter(cv, [rvec, col], val, mask=mask)

    def chunk_body(c, carry):
        rowbase = wid * _ROWS_PER_W + c * _CHUNK
        pltpu.sync_copy(x_hbm.at[pl.ds(rowbase, _CHUNK)], x_v)

        def zbody(r, carry):
            for j in range(_WP // 16):
                cnt_v[r, pl.ds(j * 16, 16)] = zero16
            return carry

        lax.fori_loop(0, _CHUNK, zbody, 0)

        def rbody(rg, carry):
            rows = [rg * 4 + i for i in range(4)]
            rvecs = [jnp.full((16,), r, jnp.int32) for r in rows]
            for j in range(_JFULL):
                xs = [x_v[r, pl.ds(j * 16, 16)] for r in rows]
                for i in range(4):
                    scat(cnt_v, xs[i], rvecs[i])
            xts = [x_v[r, pl.ds(_L - 16, 16)] for r in rows]
            for i in range(4):
                scat(cnt_v, xts[i], rvecs[i], mask=tail_mask)
            return carry

        lax.fori_loop(0, _CHUNK // 4, rbody, 0)

        pltpu.sync_copy(cnt_v, cnt_hbm.at[pl.ds(rowbase, _CHUNK)])
        return carry

    lax.fori_loop(0, _NCHUNK, chunk_body, 0)


_counts_call = pl.kernel(
    _counts_body,
    out_type=jax.ShapeDtypeStruct((_B, _WP), jnp.int32),
    mesh=plsc.VectorSubcoreMesh(core_axis_name="c", subcore_axis_name="s"),
    scratch_types=[
        pltpu.VMEM((_CHUNK, _L), jnp.int32),
        pltpu.VMEM((_CHUNK, _WP), jnp.int32),
    ],
    compiler_params=pltpu.CompilerParams(needs_layout_passes=False),
)

_BB = 512


def _mlp_body(cnt, embp, w1, b1, w2, b2, wout, bout, out, m1):
    @pl.when(pl.program_id(0) == 0)
    def _():
        m1[...] = jnp.dot(embp[...], w1[...],
                          preferred_element_type=jnp.float32) * (1.0 / _L)

    w = cnt[...]
    h = jnp.zeros((_BB, _H1), jnp.float32)
    for k in range(4):
        part = ((w >> (8 * k)) & 0xFF).astype(jnp.float32)
        h = h + jnp.dot(part, m1[pl.ds(k * _WP, _WP), :],
                        preferred_element_type=jnp.float32)
    h = jnp.maximum(h + b1[...], 0.0)
    h = jnp.maximum(
        jnp.dot(h, w2[...], preferred_element_type=jnp.float32) + b2[...], 0.0)
    out[...] = jnp.dot(h, wout[...],
                       preferred_element_type=jnp.float32) + bout[...]


_mlp_call = pl.pallas_call(
    _mlp_body,
    grid=(_B // _BB,),
    in_specs=[
        pl.BlockSpec((_BB, _WP), lambda i: (i, 0)),
        pl.BlockSpec((_VP, _EMB), lambda i: (0, 0)),
        pl.BlockSpec((_EMB, _H1), lambda i: (0, 0)),
        pl.BlockSpec((1, _H1), lambda i: (0, 0)),
        pl.BlockSpec((_H1, _H2), lambda i: (0, 0)),
        pl.BlockSpec((1, _H2), lambda i: (0, 0)),
        pl.BlockSpec((_H2, _OUT), lambda i: (0, 0)),
        pl.BlockSpec((1, _OUT), lambda i: (0, 0)),
    ],
    out_specs=pl.BlockSpec((_BB, _OUT), lambda i: (i, 0)),
    out_shape=jax.ShapeDtypeStruct((_B, _OUT), jnp.float32),
    scratch_shapes=[pltpu.VMEM((_VP, _H1), jnp.float32)],
)


def kernel(x, emb, W1, b1, W2, b2, Wout, bout):
    embp = jnp.pad(emb, ((0, _VP - _VOCAB), (0, 0)))
    cnt = _counts_call(x)
    return _mlp_call(cnt, embp, W1, b1.reshape(1, _H1), W2,
                     b2.reshape(1, _H2), Wout, bout.reshape(1, _OUT))


# speedup vs baseline: 131.4193x; 1.0327x over previous
import jax
import jax.numpy as jnp
from jax import lax
from jax.experimental import pallas as pl
from jax.experimental.pallas import tpu as pltpu
from jax.experimental.pallas import tpu_sc as plsc

_VOCAB = 1000
_VP = 1024
_EMB = 64
_B = 16384
_L = 200
_H1 = 256
_H2 = 256
_OUT = 128

_NW = 32
_ROWS_PER_W = _B // _NW
_CHUNK = 64
_NCHUNK = _ROWS_PER_W // _CHUNK
_JFULL = _L // 16


_WP = _VP // 4


def _make_counts_call(row0, nrows):
    rows_per_w = nrows // _NW
    nchunk = rows_per_w // _CHUNK

    def body(x_hbm, cnt_hbm, x_v, cnt_v):
        wid = lax.axis_index("s") * 2 + lax.axis_index("c")
        lanes = lax.iota(jnp.int32, 16)
        tail_mask = lanes >= 8
        one16 = jnp.full((16,), 1, jnp.int32)
        zero16 = jnp.zeros((16,), jnp.int32)

        def scat(cv, xv, rvec, mask=None):
            col = jnp.bitwise_and(xv, 255)
            sh = jnp.right_shift(xv, 5) & 24
            val = jnp.left_shift(one16, sh)
            plsc.addupdate_scatter(cv, [rvec, col], val, mask=mask)

        def chunk_body(c, carry):
            rowbase = wid * rows_per_w + c * _CHUNK
            pltpu.sync_copy(x_hbm.at[pl.ds(row0 + rowbase, _CHUNK)], x_v)

            def zbody(r, carry):
                for j in range(_WP // 16):
                    cnt_v[r, pl.ds(j * 16, 16)] = zero16
                return carry

            lax.fori_loop(0, _CHUNK, zbody, 0)

            def rbody(rg, carry):
                rows = [rg * 4 + i for i in range(4)]
                rvecs = [jnp.full((16,), r, jnp.int32) for r in rows]
                for j in range(_JFULL):
                    xs = [x_v[r, pl.ds(j * 16, 16)] for r in rows]
                    for i in range(4):
                        scat(cnt_v, xs[i], rvecs[i])
                xts = [x_v[r, pl.ds(_L - 16, 16)] for r in rows]
                for i in range(4):
                    scat(cnt_v, xts[i], rvecs[i], mask=tail_mask)
                return carry

            lax.fori_loop(0, _CHUNK // 4, rbody, 0)

            pltpu.sync_copy(cnt_v, cnt_hbm.at[pl.ds(rowbase, _CHUNK)])
            return carry

        lax.fori_loop(0, nchunk, chunk_body, 0)

    return pl.kernel(
        body,
        out_type=jax.ShapeDtypeStruct((nrows, _WP), jnp.int32),
        mesh=plsc.VectorSubcoreMesh(core_axis_name="c", subcore_axis_name="s"),
        scratch_types=[
            pltpu.VMEM((_CHUNK, _L), jnp.int32),
            pltpu.VMEM((_CHUNK, _WP), jnp.int32),
        ],
        compiler_params=pltpu.CompilerParams(needs_layout_passes=False),
    )


_NSPLIT = 2
_BSPLIT = _B // _NSPLIT
_counts_calls = [_make_counts_call(i * _BSPLIT, _BSPLIT)
                 for i in range(_NSPLIT)]

_BB = 512


def _mlp_body(cnt, embp, w1, b1, w2, b2, wout, bout, out, m1):
    @pl.when(pl.program_id(0) == 0)
    def _():
        m1[...] = jnp.dot(embp[...], w1[...],
                          preferred_element_type=jnp.float32) * (1.0 / _L)

    w = cnt[...]
    h = jnp.zeros((_BB, _H1), jnp.float32)
    for k in range(4):
        part = ((w >> (8 * k)) & 0xFF).astype(jnp.float32)
        h = h + jnp.dot(part, m1[pl.ds(k * _WP, _WP), :],
                        preferred_element_type=jnp.float32)
    h = jnp.maximum(h + b1[...], 0.0)
    h = jnp.maximum(
        jnp.dot(h, w2[...], preferred_element_type=jnp.float32) + b2[...], 0.0)
    out[...] = jnp.dot(h, wout[...],
                       preferred_element_type=jnp.float32) + bout[...]


_mlp_call = pl.pallas_call(
    _mlp_body,
    grid=(_BSPLIT // _BB,),
    in_specs=[
        pl.BlockSpec((_BB, _WP), lambda i: (i, 0)),
        pl.BlockSpec((_VP, _EMB), lambda i: (0, 0)),
        pl.BlockSpec((_EMB, _H1), lambda i: (0, 0)),
        pl.BlockSpec((1, _H1), lambda i: (0, 0)),
        pl.BlockSpec((_H1, _H2), lambda i: (0, 0)),
        pl.BlockSpec((1, _H2), lambda i: (0, 0)),
        pl.BlockSpec((_H2, _OUT), lambda i: (0, 0)),
        pl.BlockSpec((1, _OUT), lambda i: (0, 0)),
    ],
    out_specs=pl.BlockSpec((_BB, _OUT), lambda i: (i, 0)),
    out_shape=jax.ShapeDtypeStruct((_BSPLIT, _OUT), jnp.float32),
    scratch_shapes=[pltpu.VMEM((_VP, _H1), jnp.float32)],
)


def kernel(x, emb, W1, b1, W2, b2, Wout, bout):
    embp = jnp.pad(emb, ((0, _VP - _VOCAB), (0, 0)))
    b1r, b2r, boutr = b1.reshape(1, _H1), b2.reshape(1, _H2), bout.reshape(1, _OUT)
    cnts = [call(x) for call in _counts_calls]
    outs = [_mlp_call(c, embp, W1, b1r, W2, b2r, Wout, boutr) for c in cnts]
    return jnp.concatenate(outs, axis=0)


# speedup vs baseline: 140.2048x; 1.0669x over previous
import jax
import jax.numpy as jnp
from jax import lax
from jax.experimental import pallas as pl
from jax.experimental.pallas import tpu as pltpu
from jax.experimental.pallas import tpu_sc as plsc

_VOCAB = 1000
_VP = 1024
_EMB = 64
_B = 16384
_L = 200
_H1 = 256
_H2 = 256
_OUT = 128

_NW = 32
_ROWS_PER_W = _B // _NW
_CHUNK = 64
_NCHUNK = _ROWS_PER_W // _CHUNK
_JFULL = _L // 16


_WP = _VP // 4


def _make_counts_call(nrows):
    rows_per_w = nrows // _NW
    nchunk = rows_per_w // _CHUNK

    def body(x_hbm, cnt_hbm, x_v0, x_v1, cnt_v0, cnt_v1,
             sx0, sx1, sc0, sc1):
        wid = lax.axis_index("s") * 2 + lax.axis_index("c")
        lanes = lax.iota(jnp.int32, 16)
        tail_mask = lanes >= 8
        one16 = jnp.full((16,), 1, jnp.int32)
        zero16 = jnp.zeros((16,), jnp.int32)
        x_bufs, cnt_bufs = [x_v0, x_v1], [cnt_v0, cnt_v1]
        x_sems, cnt_sems = [sx0, sx1], [sc0, sc1]

        def scat(cv, xv, rvec, mask=None):
            col = jnp.bitwise_and(xv, 255)
            sh = jnp.right_shift(xv, 5) & 24
            val = jnp.left_shift(one16, sh)
            plsc.addupdate_scatter(cv, [rvec, col], val, mask=mask)

        def rowbase(c):
            return wid * rows_per_w + c * _CHUNK

        x_pend = [None, None]
        cnt_pend = [None, None]
        x_pend[0] = pltpu.async_copy(
            x_hbm.at[pl.ds(rowbase(0), _CHUNK)], x_bufs[0], x_sems[0])
        for c in range(nchunk):
            b = c % 2
            x_pend[b].wait()
            if c + 1 < nchunk:
                nb = (c + 1) % 2
                x_pend[nb] = pltpu.async_copy(
                    x_hbm.at[pl.ds(rowbase(c + 1), _CHUNK)],
                    x_bufs[nb], x_sems[nb])
            if cnt_pend[b] is not None:
                cnt_pend[b].wait()
            x_v, cnt_v = x_bufs[b], cnt_bufs[b]

            def zbody(r, carry):
                for j in range(_WP // 16):
                    cnt_v[r, pl.ds(j * 16, 16)] = zero16
                return carry

            lax.fori_loop(0, _CHUNK, zbody, 0)

            def rbody(rg, carry):
                rows = [rg * 4 + i for i in range(4)]
                rvecs = [jnp.full((16,), r, jnp.int32) for r in rows]
                for j in range(_JFULL):
                    xs = [x_v[r, pl.ds(j * 16, 16)] for r in rows]
                    for i in range(4):
                        scat(cnt_v, xs[i], rvecs[i])
                xts = [x_v[r, pl.ds(_L - 16, 16)] for r in rows]
                for i in range(4):
                    scat(cnt_v, xts[i], rvecs[i], mask=tail_mask)
                return carry

            lax.fori_loop(0, _CHUNK // 4, rbody, 0)

            cnt_pend[b] = pltpu.async_copy(
                cnt_v, cnt_hbm.at[pl.ds(rowbase(c), _CHUNK)], cnt_sems[b])
        for p in cnt_pend:
            if p is not None:
                p.wait()

    return pl.kernel(
        body,
        out_type=jax.ShapeDtypeStruct((nrows, _WP), jnp.int32),
        mesh=plsc.VectorSubcoreMesh(core_axis_name="c", subcore_axis_name="s"),
        scratch_types=[
            pltpu.VMEM((_CHUNK, _L), jnp.int32),
            pltpu.VMEM((_CHUNK, _L), jnp.int32),
            pltpu.VMEM((_CHUNK, _WP), jnp.int32),
            pltpu.VMEM((_CHUNK, _WP), jnp.int32),
            pltpu.SemaphoreType.DMA,
            pltpu.SemaphoreType.DMA,
            pltpu.SemaphoreType.DMA,
            pltpu.SemaphoreType.DMA,
        ],
        compiler_params=pltpu.CompilerParams(needs_layout_passes=False),
    )


_NSPLIT = 2
_BSPLIT = _B // _NSPLIT
_counts_half = _make_counts_call(_BSPLIT)

_BB = 512


def _mlp_body(cnt, embp, w1, b1, w2, b2, wout, bout, out, m1):
    @pl.when(pl.program_id(0) == 0)
    def _():
        m1[...] = jnp.dot(embp[...], w1[...],
                          preferred_element_type=jnp.float32) * (1.0 / _L)

    w = cnt[...]
    h = jnp.zeros((_BB, _H1), jnp.float32)
    for k in range(4):
        part = ((w >> (8 * k)) & 0xFF).astype(jnp.float32)
        h = h + jnp.dot(part, m1[pl.ds(k * _WP, _WP), :],
                        preferred_element_type=jnp.float32)
    h = jnp.maximum(h + b1[...], 0.0)
    h = jnp.maximum(
        jnp.dot(h, w2[...], preferred_element_type=jnp.float32) + b2[...], 0.0)
    out[...] = jnp.dot(h, wout[...],
                       preferred_element_type=jnp.float32) + bout[...]


_mlp_call = pl.pallas_call(
    _mlp_body,
    grid=(_BSPLIT // _BB,),
    in_specs=[
        pl.BlockSpec((_BB, _WP), lambda i: (i, 0)),
        pl.BlockSpec((_VP, _EMB), lambda i: (0, 0)),
        pl.BlockSpec((_EMB, _H1), lambda i: (0, 0)),
        pl.BlockSpec((1, _H1), lambda i: (0, 0)),
        pl.BlockSpec((_H1, _H2), lambda i: (0, 0)),
        pl.BlockSpec((1, _H2), lambda i: (0, 0)),
        pl.BlockSpec((_H2, _OUT), lambda i: (0, 0)),
        pl.BlockSpec((1, _OUT), lambda i: (0, 0)),
    ],
    out_specs=pl.BlockSpec((_BB, _OUT), lambda i: (i, 0)),
    out_shape=jax.ShapeDtypeStruct((_BSPLIT, _OUT), jnp.float32),
    scratch_shapes=[pltpu.VMEM((_VP, _H1), jnp.float32)],
)


def kernel(x, emb, W1, b1, W2, b2, Wout, bout):
    embp = jnp.pad(emb, ((0, _VP - _VOCAB), (0, 0)))
    b1r, b2r, boutr = b1.reshape(1, _H1), b2.reshape(1, _H2), bout.reshape(1, _OUT)
    cnts = [_counts_half(lax.slice_in_dim(x, i * _BSPLIT, (i + 1) * _BSPLIT))
            for i in range(_NSPLIT)]
    outs = [_mlp_call(c, embp, W1, b1r, W2, b2r, Wout, boutr) for c in cnts]
    return jnp.concatenate(outs, axis=0)
